# baseline (speedup 1.0000x reference)
import functools

import jax
import jax.numpy as jnp
from jax import lax
from jax.experimental import pallas as pl
from jax.experimental.pallas import tpu as pltpu
from jax.experimental.pallas import tpu_sc as plsc

_MASK_RATIO = 0.75
_WINDOW = (16, 16, 16)
_L = 16


def _sc_keep(noise_hbm, keep_hbm, noise_v, keep_v, *, B, NW, len_keep):
    NWP = noise_v.shape[0]
    nslice = NWP // _L
    wid = lax.axis_index("s") * 2 + lax.axis_index("c")

    @pl.when(wid < B)
    def _():
        b = wid
        pltpu.sync_copy(
            noise_hbm.at[pl.ds(b * NW, NW)], noise_v.at[pl.ds(0, NW)]
        )
        lanes = lax.iota(jnp.int32, _L)
        tail = noise_v[pl.ds(NWP - _L, _L)]
        tail = jnp.where(lanes + (NWP - _L) < NW, tail, jnp.inf)
        noise_v[pl.ds(NWP - _L, _L)] = tail

        def t_body(t, _):
            tv = noise_v[pl.ds(t * _L, _L)]
            widx = lanes + t * _L

            def s_body(s, rank):
                sv = noise_v[pl.ds(s * _L, _L)]

                def l_body(l, rank):
                    m = s * _L + l
                    bc = lax.gather(
                        sv,
                        jnp.full((_L, 1), l, jnp.int32),
                        lax.GatherDimensionNumbers(
                            offset_dims=(),
                            collapsed_slice_dims=(0,),
                            start_index_map=(0,),
                        ),
                        (1,),
                        mode=lax.GatherScatterMode.PROMISE_IN_BOUNDS,
                    )
                    before = (bc < tv) | ((bc == tv) & (m < widx))
                    return rank + jnp.where(before, 1, 0)

                return lax.fori_loop(0, _L, l_body, rank)

            rank = lax.fori_loop(
                0, nslice, s_body, jnp.zeros((_L,), jnp.int32)
            )
            keep_v[pl.ds(t * _L, _L)] = jnp.where(
                rank < len_keep, 1.0, 0.0
            ).astype(jnp.float32)
            return 0

        lax.fori_loop(0, nslice, t_body, 0)
        pltpu.sync_copy(
            keep_v.at[pl.ds(0, NW)], keep_hbm.at[pl.ds(b * NW, NW)]
        )


def _maskout_kernel(keep_ref, mask_ref, *, nww, nwd, W, DC):
    b = pl.program_id(0)
    i = pl.program_id(1)
    base = i * (nww * nwd)
    jwin = jax.lax.broadcasted_iota(jnp.int32, (W, DC), 0) // (W // nww)
    kwin = jax.lax.broadcasted_iota(jnp.int32, (W, DC), 1) // (DC // nwd)
    vis = jnp.zeros((W, DC), jnp.float32)
    for j in range(nww):
        for k in range(nwd):
            kv = keep_ref[b, base + j * nwd + k]
            vis = jnp.where((jwin == j) & (kwin == k), kv, vis)
    mask_ref[...] = jnp.broadcast_to(1.0 - vis[None, None], mask_ref.shape)


def _xmask_kernel(keep_ref, x_ref, xm_ref, *, nww, nwd, W, DC):
    b = pl.program_id(0)
    i = pl.program_id(1)
    base = i * (nww * nwd)
    jwin = jax.lax.broadcasted_iota(jnp.int32, (W, DC), 0) // (W // nww)
    kwin = jax.lax.broadcasted_iota(jnp.int32, (W, DC), 1) // (DC // nwd)
    vis = jnp.zeros((W, DC), jnp.float32)
    for j in range(nww):
        for k in range(nwd):
            kv = keep_ref[b, base + j * nwd + k]
            vis = jnp.where((jwin == j) & (kwin == k), kv, vis)
    xm_ref[...] = x_ref[...] * vis[None, None]


def kernel(x, noise):
    B, H, W, D, C = x.shape
    wh, ww, wd = _WINDOW
    assert H % wh == 0 and W % ww == 0 and D % wd == 0
    nwh, nww, nwd = H // wh, W // ww, D // wd
    num_windows = nwh * nww * nwd
    len_keep = int(num_windows * (1 - _MASK_RATIO))
    nwp = ((num_windows + _L - 1) // _L) * _L

    mesh = plsc.VectorSubcoreMesh(core_axis_name="c", subcore_axis_name="s")
    keep = pl.kernel(
        functools.partial(
            _sc_keep, B=B, NW=num_windows, len_keep=len_keep
        ),
        mesh=mesh,
        out_type=jax.ShapeDtypeStruct((B * num_windows,), jnp.float32),
        scratch_types=[
            pltpu.VMEM((nwp,), jnp.float32),
            pltpu.VMEM((nwp,), jnp.float32),
        ],
    )(noise.reshape(B * num_windows))
    keep = keep.reshape(B, num_windows)

    DC = D * C
    x4 = x.reshape(B, H, W, DC)
    blk = pl.BlockSpec((1, wh, W, DC), lambda b, i: (b, i, 0, 0))
    out4 = jax.ShapeDtypeStruct((B, H, W, DC), x.dtype)
    params = pltpu.CompilerParams(
        dimension_semantics=("parallel", "parallel")
    )
    mask = pl.pallas_call(
        functools.partial(_maskout_kernel, nww=nww, nwd=nwd, W=W, DC=DC),
        grid=(B, nwh),
        in_specs=[pl.BlockSpec(memory_space=pltpu.SMEM)],
        out_specs=blk,
        out_shape=out4,
        compiler_params=params,
    )(keep)
    x_masked = pl.pallas_call(
        functools.partial(_xmask_kernel, nww=nww, nwd=nwd, W=W, DC=DC),
        grid=(B, nwh),
        in_specs=[pl.BlockSpec(memory_space=pltpu.SMEM), blk],
        out_specs=blk,
        out_shape=out4,
        compiler_params=params,
    )(keep, x4)
    shape5 = (B, H, W, D, C)
    return (x_masked.reshape(shape5), mask.reshape(shape5))


# speedup vs baseline: 1.0140x; 1.0140x over previous
import functools

import jax
import jax.numpy as jnp
from jax import lax
from jax.experimental import pallas as pl
from jax.experimental.pallas import tpu as pltpu
from jax.experimental.pallas import tpu_sc as plsc

_MASK_RATIO = 0.75
_WINDOW = (16, 16, 16)
_L = 16


def _sc_keep(noise_hbm, keep_hbm, noise_v, keep_v, *, B, NW, len_keep):
    NWP = noise_v.shape[0]
    nslice = NWP // _L
    wid = lax.axis_index("s") * 2 + lax.axis_index("c")

    @pl.when(wid < B * nslice)
    def _():
        b = wid // nslice
        t = wid % nslice
        off = jnp.minimum(t * _L, NW - _L)
        pltpu.sync_copy(
            noise_hbm.at[pl.ds(b * NW, NW)], noise_v.at[pl.ds(0, NW)]
        )
        lanes = lax.iota(jnp.int32, _L)
        tail = noise_v[pl.ds(NWP - _L, _L)]
        tail = jnp.where(lanes + (NWP - _L) < NW, tail, jnp.inf)
        noise_v[pl.ds(NWP - _L, _L)] = tail

        tv = noise_v[pl.ds(off, _L)]
        widx = lanes + off

        def s_body(s, rank):
            sv = noise_v[pl.ds(s * _L, _L)]

            def l_body(l, rank):
                m = s * _L + l
                bc = lax.gather(
                    sv,
                    jnp.full((_L, 1), l, jnp.int32),
                    lax.GatherDimensionNumbers(
                        offset_dims=(),
                        collapsed_slice_dims=(0,),
                        start_index_map=(0,),
                    ),
                    (1,),
                    mode=lax.GatherScatterMode.PROMISE_IN_BOUNDS,
                )
                before = (bc < tv) | ((bc == tv) & (m < widx))
                return rank + jnp.where(before, 1, 0)

            return lax.fori_loop(0, _L, l_body, rank)

        rank = lax.fori_loop(
            0, nslice, s_body, jnp.zeros((_L,), jnp.int32)
        )
        keep_v[pl.ds(0, _L)] = jnp.where(
            rank < len_keep, 1.0, 0.0
        ).astype(jnp.float32)
        pltpu.sync_copy(
            keep_v.at[pl.ds(0, _L)], keep_hbm.at[pl.ds(b * NW + off, _L)]
        )


def _maskout_kernel(keep_ref, mask_ref, *, nww, nwd, W, DC):
    b = pl.program_id(0)
    i = pl.program_id(1)
    base = i * (nww * nwd)
    jwin = jax.lax.broadcasted_iota(jnp.int32, (W, DC), 0) // (W // nww)
    kwin = jax.lax.broadcasted_iota(jnp.int32, (W, DC), 1) // (DC // nwd)
    vis = jnp.zeros((W, DC), jnp.float32)
    for j in range(nww):
        for k in range(nwd):
            kv = keep_ref[b, base + j * nwd + k]
            vis = jnp.where((jwin == j) & (kwin == k), kv, vis)
    mask_ref[...] = jnp.broadcast_to(1.0 - vis[None, None], mask_ref.shape)


def _xmask_kernel(keep_ref, x_ref, xm_ref, *, nww, nwd, W, DC):
    b = pl.program_id(0)
    i = pl.program_id(1)
    base = i * (nww * nwd)
    jwin = jax.lax.broadcasted_iota(jnp.int32, (W, DC), 0) // (W // nww)
    kwin = jax.lax.broadcasted_iota(jnp.int32, (W, DC), 1) // (DC // nwd)
    vis = jnp.zeros((W, DC), jnp.float32)
    for j in range(nww):
        for k in range(nwd):
            kv = keep_ref[b, base + j * nwd + k]
            vis = jnp.where((jwin == j) & (kwin == k), kv, vis)
    xm_ref[...] = x_ref[...] * vis[None, None]


def kernel(x, noise):
    B, H, W, D, C = x.shape
    wh, ww, wd = _WINDOW
    assert H % wh == 0 and W % ww == 0 and D % wd == 0
    nwh, nww, nwd = H // wh, W // ww, D // wd
    num_windows = nwh * nww * nwd
    len_keep = int(num_windows * (1 - _MASK_RATIO))
    nwp = ((num_windows + _L - 1) // _L) * _L

    mesh = plsc.VectorSubcoreMesh(core_axis_name="c", subcore_axis_name="s")
    keep = pl.kernel(
        functools.partial(
            _sc_keep, B=B, NW=num_windows, len_keep=len_keep
        ),
        mesh=mesh,
        out_type=jax.ShapeDtypeStruct((B * num_windows,), jnp.float32),
        scratch_types=[
            pltpu.VMEM((nwp,), jnp.float32),
            pltpu.VMEM((nwp,), jnp.float32),
        ],
    )(noise.reshape(B * num_windows))
    keep = keep.reshape(B, num_windows)

    DC = D * C
    x4 = x.reshape(B, H, W, DC)
    blk = pl.BlockSpec((1, wh, W, DC), lambda b, i: (b, i, 0, 0))
    out4 = jax.ShapeDtypeStruct((B, H, W, DC), x.dtype)
    params = pltpu.CompilerParams(
        dimension_semantics=("parallel", "parallel")
    )
    mask = pl.pallas_call(
        functools.partial(_maskout_kernel, nww=nww, nwd=nwd, W=W, DC=DC),
        grid=(B, nwh),
        in_specs=[pl.BlockSpec(memory_space=pltpu.SMEM)],
        out_specs=blk,
        out_shape=out4,
        compiler_params=params,
    )(keep)
    x_masked = pl.pallas_call(
        functools.partial(_xmask_kernel, nww=nww, nwd=nwd, W=W, DC=DC),
        grid=(B, nwh),
        in_specs=[pl.BlockSpec(memory_space=pltpu.SMEM), blk],
        out_specs=blk,
        out_shape=out4,
        compiler_params=params,
    )(keep, x4)
    shape5 = (B, H, W, D, C)
    return (x_masked.reshape(shape5), mask.reshape(shape5))
